# Initial kernel scaffold; baseline (speedup 1.0000x reference)
import functools

import jax
import jax.numpy as jnp
from jax import lax
from jax.experimental import pallas as pl
from jax.experimental.pallas import tpu as pltpu
from jax.experimental.pallas import tpu_sc as plsc

_NC = 2
_NS = 16
_NW = _NC * _NS

_K = 128
_NBUF = 2


def _gather_kernel(idx_hbm, table_hbm, out_hbm, idx_v, rows_v, gsem, osem,
                   *, chunks_per_worker):
    wid = lax.axis_index("s") * _NC + lax.axis_index("c")
    chunk0 = wid * chunks_per_worker

    pltpu.sync_copy(idx_hbm.at[pl.ds(chunk0, chunks_per_worker)], idx_v)

    def start_gather(c, slot):
        pltpu.async_copy(table_hbm.at[idx_v.at[c]], rows_v.at[slot], gsem[slot])

    for b in range(_NBUF):
        start_gather(b, b)

    @pl.loop(0, chunks_per_worker, step=_NBUF)
    def _(c):
        for b in range(_NBUF):
            slot = b
            pltpu.make_async_copy(
                table_hbm.at[idx_v.at[c + b]], rows_v.at[slot], gsem[slot]
            ).wait()
            row0 = (chunk0 + c + b) * _K
            cp = pltpu.async_copy(
                rows_v.at[slot], out_hbm.at[pl.ds(row0, _K)], osem[slot]
            )
            nxt = c + b + _NBUF

            @pl.when(nxt < chunks_per_worker)
            def _():
                start_gather(nxt, slot)

            cp.wait()


def kernel(token_ids, weight):
    b, t = token_ids.shape
    d = weight.shape[1]
    flat_n = b * t
    assert flat_n % (_NW * _K) == 0
    chunks_per_worker = flat_n // (_NW * _K)

    idx2d = token_ids.reshape(flat_n // _K, _K).astype(jnp.int32)

    grid_kernel = functools.partial(_gather_kernel,
                                    chunks_per_worker=chunks_per_worker)
    mesh = plsc.VectorSubcoreMesh(core_axis_name="c", subcore_axis_name="s")
    out = pl.kernel(
        grid_kernel,
        out_type=jax.ShapeDtypeStruct((flat_n, d), jnp.float32),
        mesh=mesh,
        scratch_types=[
            pltpu.VMEM((chunks_per_worker, _K), jnp.int32),
            pltpu.VMEM((_NBUF, _K, d), jnp.float32),
            [pltpu.SemaphoreType.DMA] * _NBUF,
            [pltpu.SemaphoreType.DMA] * _NBUF,
        ],
    )(idx2d, weight)
    return out.reshape(b, t, d)


# speedup vs baseline: 26.1752x; 26.1752x over previous
import functools

import jax
import jax.numpy as jnp
from jax import lax
from jax.experimental import pallas as pl
from jax.experimental.pallas import tpu as pltpu
from jax.experimental.pallas import tpu_sc as plsc

_NC = 2
_NS = 16
_NW = _NC * _NS

_K = 128
_NBUF = 2


def _gather_kernel(idx_hbm, table_hbm, out_hbm, idx_v, rows_v, gsem,
                   *, chunks_per_worker):
    wid = lax.axis_index("s") * _NC + lax.axis_index("c")
    chunk0 = wid * chunks_per_worker

    pltpu.sync_copy(idx_hbm.at[wid], idx_v)

    def start_gather(c, slot):
        pltpu.async_copy(table_hbm.at[idx_v.at[c]], rows_v.at[slot], gsem[slot])

    def wait_gather(c, slot):
        pltpu.make_async_copy(
            table_hbm.at[idx_v.at[c]], rows_v.at[slot], gsem[slot]
        ).wait()

    start_gather(0, 0)
    start_gather(1, 1)

    @pl.loop(0, chunks_per_worker, step=_NBUF)
    def _(c):
        for b in range(_NBUF):
            i = c + b
            slot = b
            wait_gather(i, slot)
            row0 = (chunk0 + i) * _K
            pltpu.sync_copy(rows_v.at[slot], out_hbm.at[pl.ds(row0, _K)])
            nxt = i + _NBUF

            @pl.when(nxt < chunks_per_worker)
            def _():
                start_gather(nxt, slot)


def kernel(token_ids, weight):
    b, t = token_ids.shape
    d = weight.shape[1]
    flat_n = b * t
    assert flat_n % (_NW * _K) == 0
    chunks_per_worker = flat_n // (_NW * _K)

    idx3d = token_ids.reshape(_NW, chunks_per_worker, _K).astype(jnp.int32)

    grid_kernel = functools.partial(_gather_kernel,
                                    chunks_per_worker=chunks_per_worker)
    mesh = plsc.VectorSubcoreMesh(core_axis_name="c", subcore_axis_name="s")
    out = pl.kernel(
        grid_kernel,
        out_type=jax.ShapeDtypeStruct((flat_n, d), jnp.float32),
        mesh=mesh,
        scratch_types=[
            pltpu.VMEM((chunks_per_worker, _K), jnp.int32),
            pltpu.VMEM((_NBUF, _K, d), jnp.float32),
            [pltpu.SemaphoreType.DMA] * _NBUF,
        ],
    )(idx3d, weight)
    return out.reshape(b, t, d)


# speedup vs baseline: 26.2596x; 1.0032x over previous
import functools

import jax
import jax.numpy as jnp
from jax import lax
from jax.experimental import pallas as pl
from jax.experimental.pallas import tpu as pltpu
from jax.experimental.pallas import tpu_sc as plsc

_NC = 2
_NS = 16
_NW = _NC * _NS

_K = 128
_NBUF = 4


def _gather_kernel(idx_hbm, table_hbm, out_hbm, idx_v, rows_v, gsem, osem,
                   *, chunks_per_worker):
    n = chunks_per_worker
    wid = lax.axis_index("s") * _NC + lax.axis_index("c")
    chunk0 = wid * n

    pltpu.sync_copy(idx_hbm.at[wid], idx_v)

    def start_gather(j, s):
        pltpu.async_copy(table_hbm.at[idx_v.at[j]], rows_v.at[s], gsem[s])

    def wait_gather(j, s):
        pltpu.make_async_copy(
            table_hbm.at[idx_v.at[j]], rows_v.at[s], gsem[s]
        ).wait()

    def out_ref(j):
        return out_hbm.at[pl.ds((chunk0 + j) * _K, _K)]

    def start_out(j, s):
        pltpu.async_copy(rows_v.at[s], out_ref(j), osem[s])

    def wait_out(j, s):
        pltpu.make_async_copy(rows_v.at[s], out_ref(j), osem[s]).wait()

    start_gather(0, 0)
    start_gather(1, 1)

    def step(j, s, *, head):
        if not head:
            wait_out(j - 2, (j + 2) % _NBUF)
        start_gather(j + 2, (j + 2) % _NBUF)
        wait_gather(j, s)
        start_out(j, s)

    step(0, 0, head=True)
    step(1, 1, head=True)
    step(2, 2, head=False)
    step(3, 3, head=False)

    assert n % 4 == 2 and n >= 6

    @pl.loop(4, n - 2, step=_NBUF)
    def _(c):
        for b in range(_NBUF):
            j = c + b
            s = b
            wait_out(j - 2, (b + 2) % _NBUF)
            start_gather(j + 2, (b + 2) % _NBUF)
            wait_gather(j, s)
            start_out(j, s)

    for j in (n - 2, n - 1):
        s = j % _NBUF
        wait_gather(j, s)
        start_out(j, s)
    for j in range(n - 4, n):
        wait_out(j, j % _NBUF)


def kernel(token_ids, weight):
    b, t = token_ids.shape
    d = weight.shape[1]
    flat_n = b * t
    assert flat_n % (_NW * _K) == 0
    chunks_per_worker = flat_n // (_NW * _K)

    idx3d = token_ids.reshape(_NW, chunks_per_worker, _K).astype(jnp.int32)

    grid_kernel = functools.partial(_gather_kernel,
                                    chunks_per_worker=chunks_per_worker)
    mesh = plsc.VectorSubcoreMesh(core_axis_name="c", subcore_axis_name="s")
    out = pl.kernel(
        grid_kernel,
        out_type=jax.ShapeDtypeStruct((flat_n, d), jnp.float32),
        mesh=mesh,
        scratch_types=[
            pltpu.VMEM((chunks_per_worker, _K), jnp.int32),
            pltpu.VMEM((_NBUF, _K, d), jnp.float32),
            [pltpu.SemaphoreType.DMA] * _NBUF,
            [pltpu.SemaphoreType.DMA] * _NBUF,
        ],
    )(idx3d, weight)
    return out.reshape(b, t, d)
